# Initial kernel scaffold; baseline (speedup 1.0000x reference)
import jax
import jax.numpy as jnp
from jax.experimental import pallas as pl


def kernel(x, idx_token, agg_weight, W, b):
    raise NotImplementedError("write your pallas kernel here")


# speedup vs baseline: 8.8662x; 8.8662x over previous
import functools

import jax
import jax.numpy as jnp
from jax.experimental import pallas as pl

_B, _N, _C = 16, 2048, 128
_CL, _K = 512, 5
_TM = 256


def _dist_tile(xr, xf, sqr, sqc):
    g = jax.lax.dot_general(xr, xf, (((1,), (1,)), ((), ())),
                            preferred_element_type=jnp.float32)
    d2 = (sqr[:, None] + sqc[None, :]) - 2.0 * g
    return jnp.sqrt(jnp.maximum(d2, 0.0)) / (_C ** 0.5)


def _stage_a_body(xr_ref, xf_ref, sqr_ref, sqc_ref, w_ref,
                  nn0, nn1, nn2, nn3, nn4, rowmax_ref, ts_ref):
    xr = xr_ref[0]
    xf = xf_ref[0]
    dist = _dist_tile(xr, xf, sqr_ref[0, 0], sqc_ref[0, 0])
    rowmax_ref[0, 0] = jnp.max(dist, axis=-1)
    colid = jax.lax.broadcasted_iota(jnp.int32, (_TM, _N), 1)
    d = dist
    outs = (nn0, nn1, nn2, nn3, nn4)
    for k in range(_K):
        m = jnp.min(d, axis=-1)
        outs[k][0, 0] = m
        sel = d == m[:, None]
        ci = jnp.min(jnp.where(sel, colid, _N), axis=-1)
        d = jnp.where(colid == ci[:, None], jnp.float32(jnp.inf), d)
    ts = jax.lax.dot_general(xr, w_ref[...], (((1,), (1,)), ((), ())),
                             preferred_element_type=jnp.float32)
    ts_ref[0, 0] = ts[:, 0]


def _stage_b_body(xr_ref, xf_ref, sqr_ref, sqc_ref, den_r_ref, den_f_ref,
                  dmax_ref, score_ref):
    dist = _dist_tile(xr_ref[0], xf_ref[0], sqr_ref[0, 0], sqc_ref[0, 0])
    den_r = den_r_ref[0, 0]
    den_f = den_f_ref[0, 0]
    dmax = dmax_ref[0, 0, 0]
    mask = den_f[None, :] > den_r[:, None]
    val = jnp.where(mask, dist, dmax)
    score_ref[0, 0] = jnp.min(val, axis=-1) * den_r


def _stage_c_body(sr_ref, sf_ref, rank_ref):
    sr = sr_ref[0, 0]
    sf = sf_ref[0, 0]
    i = pl.program_id(1)
    colid = jax.lax.broadcasted_iota(jnp.int32, (_TM, _N), 1)
    rowid = i * _TM + jax.lax.broadcasted_iota(jnp.int32, (_TM, _N), 0)
    gt = (sf[None, :] > sr[:, None]).astype(jnp.int32)
    eq = ((sf[None, :] == sr[:, None]) & (colid < rowid)).astype(jnp.int32)
    rank_ref[0, 0] = jnp.sum(gt, axis=-1) + jnp.sum(eq, axis=-1)


def _stage_d_body(xr_ref, xf_ref, sqr_ref, sqc_ref, rank_r_ref, rank_f_ref,
                  idx_ref):
    dist = _dist_tile(xr_ref[0], xf_ref[0], sqr_ref[0, 0], sqc_ref[0, 0])
    rank_r = rank_r_ref[0, 0]
    rank_f = rank_f_ref[0, 0]
    is_center = rank_f < _CL
    cand = jnp.where(is_center[None, :], dist, jnp.float32(jnp.inf))
    m = jnp.min(cand, axis=-1)
    sel = cand == m[:, None]
    selrank = jnp.min(jnp.where(sel, rank_f[None, :], _CL), axis=-1)
    idx_ref[0, 0] = jnp.where(rank_r < _CL, rank_r, selrank)


def _stage_e_body(x_ref, cl_ref, tw_ref, it_ref, agg_ref,
                  xm_ref, itn_ref, awn_ref):
    cl = cl_ref[0, 0]
    tw = tw_ref[0, 0]
    clid = jax.lax.broadcasted_iota(jnp.int32, (_CL, _N), 0)
    oh = (clid == cl[None, :]).astype(jnp.float32)
    aw = jnp.sum(oh * tw[None, :], axis=-1) + 1e-06
    awg = jnp.sum(oh * aw[:, None], axis=0)
    norm = tw / awg
    src = x_ref[0] * norm[:, None]
    xm_ref[0] = jax.lax.dot_general(
        oh, src, (((1,), (0,)), ((), ())),
        precision=jax.lax.Precision.HIGHEST, preferred_element_type=jnp.float32)
    it = it_ref[0, 0]
    colid = jax.lax.broadcasted_iota(jnp.int32, (_TM, _N), 1)
    clf = cl[None, :]
    nrm = norm[None, :]
    for t in range(_N // _TM):
        itc = it[t * _TM:(t + 1) * _TM]
        ohg = itc[:, None] == colid
        gcl = jnp.sum(jnp.where(ohg, clf, 0), axis=-1)
        gnw = jnp.sum(jnp.where(ohg, nrm, 0.0), axis=-1)
        itn_ref[0, 0, t * _TM:(t + 1) * _TM] = gcl
        awn_ref[0, 0, t * _TM:(t + 1) * _TM] = (
            agg_ref[0, 0, t * _TM:(t + 1) * _TM] * gnw)


def _row_spec():
    return pl.BlockSpec((1, _TM, _C), lambda b, i: (b, i, 0))


def _full_spec():
    return pl.BlockSpec((1, _N, _C), lambda b, i: (b, 0, 0))


def _vec_r_spec():
    return pl.BlockSpec((1, 1, _TM), lambda b, i: (b, 0, i))


def _vec_f_spec():
    return pl.BlockSpec((1, 1, _N), lambda b, i: (b, 0, 0))


def kernel(x, idx_token, agg_weight, W, b):
    B, N, C = x.shape
    grid = (B, N // _TM)
    f32 = jnp.float32
    vec = jax.ShapeDtypeStruct((B, 1, N), f32)

    sq = jnp.sum(x * x, axis=-1)[:, None, :]

    nn = [vec] * _K
    nn0, nn1, nn2, nn3, nn4, rowmax, ts = pl.pallas_call(
        _stage_a_body,
        grid=grid,
        in_specs=[_row_spec(), _full_spec(), _vec_r_spec(), _vec_f_spec(),
                  pl.BlockSpec((1, _C), lambda b, i: (0, 0))],
        out_specs=[_vec_r_spec()] * (_K + 2),
        out_shape=nn + [vec, vec],
    )(x, x, sq, sq, W)

    dist_nearest = jnp.concatenate(
        [o.reshape(B, N, 1) for o in (nn0, nn1, nn2, nn3, nn4)], axis=-1)
    density = jnp.exp(-jnp.mean(dist_nearest ** 2, axis=-1))
    noise = jax.random.uniform(jax.random.key(1), density.shape,
                               dtype=density.dtype) * 1e-06
    density = (density + noise)[:, None, :]
    dist_max = jnp.max(rowmax, axis=-1, keepdims=True)
    token_weight = jnp.exp(ts + b[0])

    score = pl.pallas_call(
        _stage_b_body,
        grid=grid,
        in_specs=[_row_spec(), _full_spec(), _vec_r_spec(), _vec_f_spec(),
                  _vec_r_spec(), _vec_f_spec(),
                  pl.BlockSpec((1, 1, 1), lambda b, i: (b, 0, 0))],
        out_specs=_vec_r_spec(),
        out_shape=vec,
    )(x, x, sq, sq, density, density, dist_max)

    rank = pl.pallas_call(
        _stage_c_body,
        grid=grid,
        in_specs=[_vec_r_spec(), _vec_f_spec()],
        out_specs=_vec_r_spec(),
        out_shape=jax.ShapeDtypeStruct((B, 1, N), jnp.int32),
    )(score, score)

    idx_cluster = pl.pallas_call(
        _stage_d_body,
        grid=grid,
        in_specs=[_row_spec(), _full_spec(), _vec_r_spec(), _vec_f_spec(),
                  _vec_r_spec(), _vec_f_spec()],
        out_specs=_vec_r_spec(),
        out_shape=jax.ShapeDtypeStruct((B, 1, N), jnp.int32),
    )(x, x, sq, sq, rank, rank)

    x_merged, idx_token_new, agg_weight_new = pl.pallas_call(
        _stage_e_body,
        grid=(B,),
        in_specs=[pl.BlockSpec((1, N, C), lambda b: (b, 0, 0)),
                  pl.BlockSpec((1, 1, N), lambda b: (b, 0, 0)),
                  pl.BlockSpec((1, 1, N), lambda b: (b, 0, 0)),
                  pl.BlockSpec((1, 1, N), lambda b: (b, 0, 0)),
                  pl.BlockSpec((1, 1, N), lambda b: (b, 0, 0))],
        out_specs=[pl.BlockSpec((1, _CL, C), lambda b: (b, 0, 0)),
                   pl.BlockSpec((1, 1, N), lambda b: (b, 0, 0)),
                   pl.BlockSpec((1, 1, N), lambda b: (b, 0, 0))],
        out_shape=[jax.ShapeDtypeStruct((B, _CL, C), f32),
                   jax.ShapeDtypeStruct((B, 1, N), jnp.int32),
                   jax.ShapeDtypeStruct((B, 1, N), f32)],
    )(x, idx_cluster, token_weight, idx_token.reshape(B, 1, N),
      agg_weight.reshape(B, 1, N))

    return (x_merged,
            idx_token_new.reshape(B, N),
            agg_weight_new.reshape(B, N, 1))
